# baseline (speedup 1.0000x reference)
import functools

import jax
import jax.numpy as jnp
from jax import lax
from jax.experimental import pallas as pl
from jax.experimental.pallas import tpu as pltpu
from jax.experimental.pallas import tpu_sc as plsc

NC = 2
NS = 16
L = 16
NW = NC * NS
IDX_CHUNK = 128
TBLK = 32768
QBLK = TBLK // 4
SH_T = TBLK.bit_length() - 1
SH_Q = QBLK.bit_length() - 1
SC_CHUNK = 256


def _relayout_inner(ut_v, it_v, u_v, i_v):
    for src, dst in ((ut_v, u_v), (it_v, i_v)):
        x = src[...]
        x2 = x.reshape(32, 4, QBLK).transpose(1, 0, 2).reshape(128, QBLK)
        dst[...] = x2.T


@functools.partial(jax.jit, static_argnames=("B", "D"))
def _mf_dot(user_idx, item_idx, uemb_t, iemb_t, *, B, D):
    V = uemb_t.shape[1]
    bpw = B // NW
    nchunk = bpw // IDX_CHUNK
    nblocks = (V + TBLK - 1) // TBLK
    vpad = nblocks * QBLK

    tc_mesh = pltpu.create_tensorcore_mesh("x", num_cores=2)

    def _relayout_outer(ut_ref, it_ref, u_ref, i_ref):
        pltpu.emit_pipeline(
            _relayout_inner,
            grid=(nblocks,),
            in_specs=[
                pl.BlockSpec((D, TBLK), lambda j: (0, j)),
                pl.BlockSpec((D, TBLK), lambda j: (0, j)),
            ],
            out_specs=[
                pl.BlockSpec((QBLK, 128), lambda j: (j, 0)),
                pl.BlockSpec((QBLK, 128), lambda j: (j, 0)),
            ],
            core_axis_name="x",
            dimension_semantics=(pltpu.PARALLEL,),
        )(ut_ref, it_ref, u_ref, i_ref)

    u_rm, i_rm = pl.kernel(
        _relayout_outer,
        out_type=[
            jax.ShapeDtypeStruct((vpad, 128), jnp.float32),
            jax.ShapeDtypeStruct((vpad, 128), jnp.float32),
        ],
        mesh=tc_mesh,
    )(uemb_t, iemb_t)

    mesh = plsc.VectorSubcoreMesh(
        core_axis_name="c", subcore_axis_name="s", num_cores=NC, num_subcores=NS
    )
    cparams = pltpu.CompilerParams(
        needs_layout_passes=False, use_tc_tiling_on_sc=False
    )

    @functools.partial(
        pl.kernel,
        mesh=mesh,
        compiler_params=cparams,
        out_type=jax.ShapeDtypeStruct((B,), jnp.float32),
        scratch_types=[
            pltpu.VMEM((nchunk, IDX_CHUNK), jnp.int32),
            pltpu.VMEM((nchunk, IDX_CHUNK), jnp.int32),
            pltpu.VMEM((nchunk, IDX_CHUNK), jnp.int32),
            pltpu.VMEM((nchunk, IDX_CHUNK), jnp.int32),
            pltpu.VMEM((SC_CHUNK, 128), jnp.float32),
            pltpu.VMEM((SC_CHUNK, 128), jnp.float32),
            pltpu.VMEM((bpw,), jnp.float32),
            pltpu.SemaphoreType.DMA,
        ],
    )
    def k(uemb_hbm, iemb_hbm, uidx_hbm, iidx_hbm, out_hbm,
          idxu, idxi, rowu, rowi, urows, vrows, outv, sem):
        wid = lax.axis_index("s") * NC + lax.axis_index("c")
        base = wid * bpw
        pltpu.sync_copy(uidx_hbm.at[wid], idxu)
        pltpu.sync_copy(iidx_hbm.at[wid], idxi)

        for src, dst in ((idxu, rowu), (idxi, rowi)):
            @pl.loop(0, nchunk)
            def _(j):
                @pl.loop(0, IDX_CHUNK // L)
                def _(t):
                    sl = pl.ds(t * L, L)
                    u = src[j, sl]
                    r = (
                        lax.shift_left(lax.shift_right_logical(u, SH_T), SH_Q)
                        + lax.bitwise_and(u, QBLK - 1)
                    )
                    dst[j, sl] = r

        ngrp = SC_CHUNK // L
        half = bpw // SC_CHUNK

        @pl.loop(0, half)
        def _(c):
            copies = []
            for j in range(SC_CHUNK // IDX_CHUNK):
                jj = c * (SC_CHUNK // IDX_CHUNK) + j
                dst = urows.at[pl.ds(j * IDX_CHUNK, IDX_CHUNK)]
                copies.append(
                    pltpu.async_copy(uemb_hbm.at[rowu.at[jj]], dst, sem)
                )
                dst = vrows.at[pl.ds(j * IDX_CHUNK, IDX_CHUNK)]
                copies.append(
                    pltpu.async_copy(iemb_hbm.at[rowi.at[jj]], dst, sem)
                )
            for cp in copies:
                cp.wait()

            iota = lax.iota(jnp.int32, L)

            @pl.loop(0, ngrp)
            def _(g):
                row = g * L + iota
                pos = c * SC_CHUNK + g * L
                uvec = idxu[pos // IDX_CHUNK, pl.ds(pos % IDX_CHUNK, L)]
                ivec = idxi[pos // IDX_CHUNK, pl.ds(pos % IDX_CHUNK, L)]
                uoff = lax.shift_left(
                    lax.bitwise_and(lax.shift_right_logical(uvec, SH_Q), 3), 5
                )
                ioff = lax.shift_left(
                    lax.bitwise_and(lax.shift_right_logical(ivec, SH_Q), 3), 5
                )
                acc = jnp.zeros((L,), jnp.float32)
                for e in range(D):
                    col = iota + e
                    col = jnp.where(col >= D, col - D, col)
                    uu = plsc.load_gather(urows, [row, uoff + col])
                    vv = plsc.load_gather(vrows, [row, ioff + col])
                    acc = acc + uu * vv
                outv[pl.ds(pos, L)] = acc

        pltpu.sync_copy(outv, out_hbm.at[pl.ds(base, bpw)])

    return k(u_rm, i_rm, user_idx, item_idx)


def kernel(user, item, user_emb, item_emb):
    B = user.shape[0]
    D = user_emb.shape[1]
    bpw = B // NW
    nchunk = bpw // IDX_CHUNK
    uidx = user.astype(jnp.int32).reshape(NW, nchunk, IDX_CHUNK)
    iidx = item.astype(jnp.int32).reshape(NW, nchunk, IDX_CHUNK)
    return _mf_dot(uidx, iidx, user_emb.T, item_emb.T, B=B, D=D)


# speedup vs baseline: 1.0080x; 1.0080x over previous
import functools

import jax
import jax.numpy as jnp
from jax import lax
from jax.experimental import pallas as pl
from jax.experimental.pallas import tpu as pltpu
from jax.experimental.pallas import tpu_sc as plsc

NC = 2
NS = 16
L = 16
NW = NC * NS
IDX_CHUNK = 128
TBLK = 32768
QBLK = TBLK // 4
SH_T = TBLK.bit_length() - 1
SH_Q = QBLK.bit_length() - 1
SC_CHUNK = 256


def _relayout_body(ut_ref, it_ref, u_ref, i_ref):
    for src, dst in ((ut_ref, u_ref), (it_ref, i_ref)):
        x = src[...]
        x2 = x.reshape(32, 4, QBLK).transpose(1, 0, 2).reshape(128, QBLK)
        dst[...] = x2.T


@functools.partial(jax.jit, static_argnames=("B", "D"))
def _mf_dot(user_idx, item_idx, uemb_t, iemb_t, *, B, D):
    V = uemb_t.shape[1]
    bpw = B // NW
    nchunk = bpw // IDX_CHUNK
    nblocks = (V + TBLK - 1) // TBLK
    vpad = nblocks * QBLK

    u_rm, i_rm = pl.pallas_call(
        _relayout_body,
        grid=(nblocks,),
        in_specs=[
            pl.BlockSpec((D, TBLK), lambda j: (0, j)),
            pl.BlockSpec((D, TBLK), lambda j: (0, j)),
        ],
        out_specs=[
            pl.BlockSpec((QBLK, 128), lambda j: (j, 0)),
            pl.BlockSpec((QBLK, 128), lambda j: (j, 0)),
        ],
        out_shape=[
            jax.ShapeDtypeStruct((vpad, 128), jnp.float32),
            jax.ShapeDtypeStruct((vpad, 128), jnp.float32),
        ],
        compiler_params=pltpu.CompilerParams(
            dimension_semantics=("parallel",)
        ),
    )(uemb_t, iemb_t)

    mesh = plsc.VectorSubcoreMesh(
        core_axis_name="c", subcore_axis_name="s", num_cores=NC, num_subcores=NS
    )
    cparams = pltpu.CompilerParams(
        needs_layout_passes=False, use_tc_tiling_on_sc=False
    )

    @functools.partial(
        pl.kernel,
        mesh=mesh,
        compiler_params=cparams,
        out_type=jax.ShapeDtypeStruct((B,), jnp.float32),
        scratch_types=[
            pltpu.VMEM((nchunk, IDX_CHUNK), jnp.int32),
            pltpu.VMEM((nchunk, IDX_CHUNK), jnp.int32),
            pltpu.VMEM((nchunk, IDX_CHUNK), jnp.int32),
            pltpu.VMEM((nchunk, IDX_CHUNK), jnp.int32),
            pltpu.VMEM((SC_CHUNK, 128), jnp.float32),
            pltpu.VMEM((SC_CHUNK, 128), jnp.float32),
            pltpu.VMEM((bpw,), jnp.float32),
            pltpu.SemaphoreType.DMA,
        ],
    )
    def k(uemb_hbm, iemb_hbm, uidx_hbm, iidx_hbm, out_hbm,
          idxu, idxi, rowu, rowi, urows, vrows, outv, sem):
        wid = lax.axis_index("s") * NC + lax.axis_index("c")
        base = wid * bpw
        pltpu.sync_copy(uidx_hbm.at[wid], idxu)
        pltpu.sync_copy(iidx_hbm.at[wid], idxi)

        for src, dst in ((idxu, rowu), (idxi, rowi)):
            @pl.loop(0, nchunk)
            def _(j):
                @pl.loop(0, IDX_CHUNK // L)
                def _(t):
                    sl = pl.ds(t * L, L)
                    u = src[j, sl]
                    r = (
                        lax.shift_left(lax.shift_right_logical(u, SH_T), SH_Q)
                        + lax.bitwise_and(u, QBLK - 1)
                    )
                    dst[j, sl] = r

        ngrp = SC_CHUNK // L
        half = bpw // SC_CHUNK

        @pl.loop(0, half)
        def _(c):
            copies = []
            for j in range(SC_CHUNK // IDX_CHUNK):
                jj = c * (SC_CHUNK // IDX_CHUNK) + j
                dst = urows.at[pl.ds(j * IDX_CHUNK, IDX_CHUNK)]
                copies.append(
                    pltpu.async_copy(uemb_hbm.at[rowu.at[jj]], dst, sem)
                )
                dst = vrows.at[pl.ds(j * IDX_CHUNK, IDX_CHUNK)]
                copies.append(
                    pltpu.async_copy(iemb_hbm.at[rowi.at[jj]], dst, sem)
                )
            for cp in copies:
                cp.wait()

            iota = lax.iota(jnp.int32, L)

            @pl.loop(0, ngrp)
            def _(g):
                row = g * L + iota
                pos = c * SC_CHUNK + g * L
                uvec = idxu[pos // IDX_CHUNK, pl.ds(pos % IDX_CHUNK, L)]
                ivec = idxi[pos // IDX_CHUNK, pl.ds(pos % IDX_CHUNK, L)]
                uoff = lax.shift_left(
                    lax.bitwise_and(lax.shift_right_logical(uvec, SH_Q), 3), 5
                )
                ioff = lax.shift_left(
                    lax.bitwise_and(lax.shift_right_logical(ivec, SH_Q), 3), 5
                )
                acc = jnp.zeros((L,), jnp.float32)
                for e in range(D):
                    col = iota + e
                    col = jnp.where(col >= D, col - D, col)
                    uu = plsc.load_gather(urows, [row, uoff + col])
                    vv = plsc.load_gather(vrows, [row, ioff + col])
                    acc = acc + uu * vv
                outv[pl.ds(pos, L)] = acc

        pltpu.sync_copy(outv, out_hbm.at[pl.ds(base, bpw)])

    return k(u_rm, i_rm, user_idx, item_idx)


def kernel(user, item, user_emb, item_emb):
    B = user.shape[0]
    D = user_emb.shape[1]
    bpw = B // NW
    nchunk = bpw // IDX_CHUNK
    uidx = user.astype(jnp.int32).reshape(NW, nchunk, IDX_CHUNK)
    iidx = item.astype(jnp.int32).reshape(NW, nchunk, IDX_CHUNK)
    return _mf_dot(uidx, iidx, user_emb.T, item_emb.T, B=B, D=D)
